# baseline (speedup 1.0000x reference)
import dataclasses
import functools

import numpy as np

import jax
import jax.numpy as jnp
from jax import lax
from jax.experimental import pallas as pl
from jax.experimental.pallas import tpu as pltpu
from jax.experimental.pallas import tpu_sc as plsc

_N = 10000
_E = 320000
_D = 128
_NC = 2
_NS = 16
_NW = _NC * _NS
_EB = 128
_BPW = 80
_NBLK = _NW * _BPW
_EPAD = _NBLK * _EB
_NPAD = 10240
_RPS = _NPAD // _NS
_MBLK = 5000

_NRBLK = _E // _EB
_PAD3 = np.stack(
    [
        (np.arange(_EPAD - _E, dtype=np.int32) % 8192).reshape(_NBLK - _NRBLK, _EB),
        (_N + np.arange(_EPAD - _E, dtype=np.int32) % 240).reshape(_NBLK - _NRBLK, _EB),
    ],
    axis=1,
)

_mesh = plsc.VectorSubcoreMesh(
    core_axis_name="c", subcore_axis_name="s", num_cores=_NC, num_subcores=_NS
)


_DIBC = 16
_DNCH = _BPW // _DIBC


def _deg_body(dst_hbm, tail_hbm, out_hbm, dstv, hist_v, red_v, out16_v, sh):
    c = lax.axis_index("c")
    s = lax.axis_index("s")
    wid = s * _NC + c

    @pl.loop(0, _NPAD, step=16)
    def _(k):
        hist_v[pl.ds(k, 16)] = jnp.zeros((16,), jnp.int32)

    def hist_slab(src_ref, base):
        @pl.loop(0, _DNCH)
        def _(ci):
            pltpu.sync_copy(src_ref.at[pl.ds(base + ci * _DIBC, _DIBC)], dstv)

            @pl.loop(0, _DIBC)
            def _(j):
                for k in range(_EB // 16):
                    idx = dstv[j, 1, pl.ds(k * 16, 16)]
                    cnt, last = plsc.scan_count(idx)
                    plsc.addupdate_scatter(hist_v, [idx], cnt, mask=last)

    @pl.when(wid < _NW - 1)
    def _():
        hist_slab(dst_hbm, wid * _BPW)

    @pl.when(wid == _NW - 1)
    def _():
        hist_slab(tail_hbm, 0)

    pltpu.sync_copy(hist_v, sh.at[s])
    plsc.subcore_barrier()
    for k in range(_NS):
        pltpu.sync_copy(sh.at[k].at[pl.ds(s * _RPS, _RPS)], red_v.at[k])

    @pl.loop(0, _RPS, step=16)
    def _(l):
        acc = jnp.zeros((16,), jnp.int32)
        for k in range(_NS):
            acc = acc + red_v[k, pl.ds(l, 16)]
        cf = acc.astype(jnp.float32)
        for i in range(16):
            out16_v[l + i, :] = jnp.full((16,), cf[i], jnp.float32)

    pltpu.sync_copy(out16_v, out_hbm.at[c].at[pl.ds(s * _RPS, _RPS)])


_cp = pltpu.CompilerParams()
if "needs_layout_passes" in pltpu.CompilerParams.__dataclass_fields__:
    _cp = dataclasses.replace(_cp, needs_layout_passes=False)

_deg = functools.partial(
    pl.kernel,
    out_type=jax.ShapeDtypeStruct((_NC, _NPAD, 16), jnp.float32),
    compiler_params=_cp,
    mesh=_mesh,
    scratch_types=[
        pltpu.VMEM((_DIBC, 2, _EB), jnp.int32),
        pltpu.VMEM((_NPAD,), jnp.int32),
        pltpu.VMEM((_NS, _RPS), jnp.int32),
        pltpu.VMEM((_RPS, 16), jnp.float32),
        pltpu.VMEM_SHARED((_NS, _NPAD), jnp.int32),
    ],
)(_deg_body)


_NBUF = 2
_IBC = 16
_NCH = _BPW // _IBC


def _agg_body(
    hs_hbm, eidx_hbm, tail_hbm, zero_hbm, out_hbm,
    ev, rows0, rows1, sem0, sem1, acc_sh
):
    c = lax.axis_index("c")
    s = lax.axis_index("s")
    wid = s * _NC + c
    pltpu.sync_copy(zero_hbm, acc_sh.at[pl.ds(s * _RPS, _RPS)])
    plsc.subcore_barrier()

    bufs = (rows0, rows1)
    sems = (sem0, sem1)

    def agg_slab(src_ref, base):
        @pl.loop(0, _NCH)
        def _(ci):
            cb = base + ci * _IBC
            pltpu.sync_copy(src_ref.at[pl.ds(cb, _IBC)], ev)
            for b in range(_NBUF):
                pltpu.async_copy(hs_hbm.at[ev.at[b, 0]], bufs[b], sems[b])

            @pl.loop(0, _IBC, step=_NBUF)
            def _(j):
                for b in range(_NBUF):
                    jb = j + b
                    pltpu.make_async_copy(
                        hs_hbm.at[ev.at[jb, 0]], bufs[b], sems[b]
                    ).wait()
                    pltpu.sync_copy(bufs[b], acc_sh.at[ev.at[jb, 1]], add=True)

                    @pl.when(jb + _NBUF < _IBC)
                    def _():
                        pltpu.async_copy(
                            hs_hbm.at[ev.at[jb + _NBUF, 0]], bufs[b], sems[b]
                        )

    @pl.when(wid < _NW - 1)
    def _():
        agg_slab(eidx_hbm, wid * _BPW)

    @pl.when(wid == _NW - 1)
    def _():
        agg_slab(tail_hbm, 0)

    plsc.subcore_barrier()
    pltpu.sync_copy(
        acc_sh.at[pl.ds(s * _RPS, _RPS)],
        out_hbm.at[c].at[pl.ds(s * _RPS, _RPS)],
    )


_agg = functools.partial(
    pl.kernel,
    out_type=jax.ShapeDtypeStruct((_NC, _NPAD, _D), jnp.float32),
    mesh=_mesh,
    scratch_types=[
        pltpu.VMEM((_IBC, 2, _EB), jnp.int32),
        pltpu.VMEM((_EB, _D), jnp.float32),
        pltpu.VMEM((_EB, _D), jnp.float32),
        pltpu.SemaphoreType.DMA,
        pltpu.SemaphoreType.DMA,
        pltpu.VMEM_SHARED((_NPAD, _D), jnp.float32),
    ],
)(_agg_body)


def _dot(a, b):
    return lax.dot_general(
        a,
        b,
        (((1,), (0,)), ((), ())),
        precision=lax.Precision.HIGHEST,
        preferred_element_type=jnp.float32,
    )


def _dinv_col(degp_ref):
    deg = degp_ref[0, :, 0:1] + degp_ref[1, :, 0:1] + 1.0
    return lax.rsqrt(deg)


def _mm_body(x_ref, w_ref, o_ref):
    o_ref[...] = _dot(x_ref[...], w_ref[...])


def _scale_body(h_ref, degp_ref, hs_ref):
    hs_ref[...] = h_ref[...] * _dinv_col(degp_ref)


def _combine_mm_body(accp_ref, hs_ref, degp_ref, b_ref, w_ref, o_ref):
    d = _dinv_col(degp_ref)
    z = d * (accp_ref[0] + accp_ref[1] + hs_ref[...]) + b_ref[...]
    z = jnp.maximum(z, 0.0)
    o_ref[...] = _dot(z, w_ref[...]) * d


def _final_body(accp_ref, hs_ref, degp_ref, b_ref, o_ref):
    d = _dinv_col(degp_ref)
    o_ref[...] = d * (accp_ref[0] + accp_ref[1] + hs_ref[...]) + b_ref[...]


def kernel(x, edge_idx, W1, b1, W2, b2):
    n, d = x.shape
    grid = (n // _MBLK,)
    ei3 = jnp.transpose(edge_idx.reshape(2, _NRBLK, _EB), (1, 0, 2))
    tail = jnp.concatenate([ei3[(_NW - 1) * _BPW :], jnp.asarray(_PAD3)])
    zeroD = jnp.zeros((_RPS, _D), jnp.float32)

    degp = _deg(ei3, tail)

    h1 = pl.pallas_call(
        _mm_body,
        grid=grid,
        in_specs=[
            pl.BlockSpec((_MBLK, d), lambda i: (i, 0)),
            pl.BlockSpec((d, d), lambda i: (0, 0)),
        ],
        out_specs=pl.BlockSpec((_MBLK, d), lambda i: (i, 0)),
        out_shape=jax.ShapeDtypeStruct((n, d), jnp.float32),
    )(x, W1)

    hs1 = pl.pallas_call(
        _scale_body,
        grid=grid,
        in_specs=[
            pl.BlockSpec((_MBLK, d), lambda i: (i, 0)),
            pl.BlockSpec((_NC, _MBLK, 16), lambda i: (0, i, 0)),
        ],
        out_specs=pl.BlockSpec((_MBLK, d), lambda i: (i, 0)),
        out_shape=jax.ShapeDtypeStruct((n, d), jnp.float32),
    )(h1, degp)

    acc1 = _agg(hs1, ei3, tail, zeroD)

    hs2 = pl.pallas_call(
        _combine_mm_body,
        grid=grid,
        in_specs=[
            pl.BlockSpec((_NC, _MBLK, _D), lambda i: (0, i, 0)),
            pl.BlockSpec((_MBLK, d), lambda i: (i, 0)),
            pl.BlockSpec((_NC, _MBLK, 16), lambda i: (0, i, 0)),
            pl.BlockSpec((1, d), lambda i: (0, 0)),
            pl.BlockSpec((d, d), lambda i: (0, 0)),
        ],
        out_specs=pl.BlockSpec((_MBLK, d), lambda i: (i, 0)),
        out_shape=jax.ShapeDtypeStruct((n, d), jnp.float32),
    )(acc1, hs1, degp, b1.reshape(1, d), W2)

    acc2 = _agg(hs2, ei3, tail, zeroD)

    out = pl.pallas_call(
        _final_body,
        grid=grid,
        in_specs=[
            pl.BlockSpec((_NC, _MBLK, _D), lambda i: (0, i, 0)),
            pl.BlockSpec((_MBLK, d), lambda i: (i, 0)),
            pl.BlockSpec((_NC, _MBLK, 16), lambda i: (0, i, 0)),
            pl.BlockSpec((1, d), lambda i: (0, 0)),
        ],
        out_specs=pl.BlockSpec((_MBLK, d), lambda i: (i, 0)),
        out_shape=jax.ShapeDtypeStruct((n, d), jnp.float32),
    )(acc2, hs2, degp, b2.reshape(1, d))

    return out


# speedup vs baseline: 1.0336x; 1.0336x over previous
import dataclasses
import functools

import numpy as np

import jax
import jax.numpy as jnp
from jax import lax
from jax.experimental import pallas as pl
from jax.experimental.pallas import tpu as pltpu
from jax.experimental.pallas import tpu_sc as plsc

_N = 10000
_E = 320000
_D = 128
_NC = 2
_NS = 16
_NW = _NC * _NS
_EB = 128
_BPW = 80
_NBLK = _NW * _BPW
_EPAD = _NBLK * _EB
_NPAD = 10240
_RPS = _NPAD // _NS
_MBLK = 2000

_NRBLK = _E // _EB
_PAD3 = np.stack(
    [
        (np.arange(_EPAD - _E, dtype=np.int32) % 8192).reshape(_NBLK - _NRBLK, _EB),
        (_N + np.arange(_EPAD - _E, dtype=np.int32) % 240).reshape(_NBLK - _NRBLK, _EB),
    ],
    axis=1,
)

_mesh = plsc.VectorSubcoreMesh(
    core_axis_name="c", subcore_axis_name="s", num_cores=_NC, num_subcores=_NS
)


_DIBC = 16
_DNCH = _BPW // _DIBC


def _deg_body(dst_hbm, tail_hbm, out_hbm, dstv, hist_v, red_v, out16_v, sh):
    c = lax.axis_index("c")
    s = lax.axis_index("s")
    wid = s * _NC + c

    @pl.loop(0, _NPAD, step=16)
    def _(k):
        hist_v[pl.ds(k, 16)] = jnp.zeros((16,), jnp.int32)

    def hist_slab(src_ref, base):
        @pl.loop(0, _DNCH)
        def _(ci):
            pltpu.sync_copy(src_ref.at[pl.ds(base + ci * _DIBC, _DIBC)], dstv)

            @pl.loop(0, _DIBC)
            def _(j):
                for k in range(_EB // 16):
                    idx = dstv[j, 1, pl.ds(k * 16, 16)]
                    cnt, last = plsc.scan_count(idx)
                    plsc.addupdate_scatter(hist_v, [idx], cnt, mask=last)

    @pl.when(wid < _NW - 1)
    def _():
        hist_slab(dst_hbm, wid * _BPW)

    @pl.when(wid == _NW - 1)
    def _():
        hist_slab(tail_hbm, 0)

    pltpu.sync_copy(hist_v, sh.at[s])
    plsc.subcore_barrier()
    for k in range(_NS):
        pltpu.sync_copy(sh.at[k].at[pl.ds(s * _RPS, _RPS)], red_v.at[k])

    @pl.loop(0, _RPS, step=16)
    def _(l):
        acc = jnp.zeros((16,), jnp.int32)
        for k in range(_NS):
            acc = acc + red_v[k, pl.ds(l, 16)]
        cf = acc.astype(jnp.float32)
        for i in range(16):
            out16_v[l + i, :] = jnp.full((16,), cf[i], jnp.float32)

    pltpu.sync_copy(out16_v, out_hbm.at[c].at[pl.ds(s * _RPS, _RPS)])


_cp = pltpu.CompilerParams()
if "needs_layout_passes" in pltpu.CompilerParams.__dataclass_fields__:
    _cp = dataclasses.replace(_cp, needs_layout_passes=False)

_deg = functools.partial(
    pl.kernel,
    out_type=jax.ShapeDtypeStruct((_NC, _NPAD, 16), jnp.float32),
    compiler_params=_cp,
    mesh=_mesh,
    scratch_types=[
        pltpu.VMEM((_DIBC, 2, _EB), jnp.int32),
        pltpu.VMEM((_NPAD,), jnp.int32),
        pltpu.VMEM((_NS, _RPS), jnp.int32),
        pltpu.VMEM((_RPS, 16), jnp.float32),
        pltpu.VMEM_SHARED((_NS, _NPAD), jnp.int32),
    ],
)(_deg_body)


_NBUF = 2
_IBC = 16
_NCH = _BPW // _IBC


def _agg_body(
    hs_hbm, eidx_hbm, tail_hbm, out_hbm,
    ev, rows0, rows1, sem0, sem1, acc_sh
):
    c = lax.axis_index("c")
    s = lax.axis_index("s")
    wid = s * _NC + c

    @pl.loop(0, _EB)
    def _(i):
        for k in range(_D // 16):
            rows0[i, pl.ds(k * 16, 16)] = jnp.zeros((16,), jnp.float32)

    @pl.loop(0, _RPS // _EB)
    def _(i):
        pltpu.sync_copy(rows0, acc_sh.at[pl.ds(s * _RPS + i * _EB, _EB)])

    plsc.subcore_barrier()

    bufs = (rows0, rows1)
    sems = (sem0, sem1)

    def agg_slab(src_ref, base):
        @pl.loop(0, _NCH)
        def _(ci):
            cb = base + ci * _IBC
            pltpu.sync_copy(src_ref.at[pl.ds(cb, _IBC)], ev)
            for b in range(_NBUF):
                pltpu.async_copy(hs_hbm.at[ev.at[b, 0]], bufs[b], sems[b])

            @pl.loop(0, _IBC, step=_NBUF)
            def _(j):
                for b in range(_NBUF):
                    jb = j + b
                    pltpu.make_async_copy(
                        hs_hbm.at[ev.at[jb, 0]], bufs[b], sems[b]
                    ).wait()
                    pltpu.sync_copy(bufs[b], acc_sh.at[ev.at[jb, 1]], add=True)

                    @pl.when(jb + _NBUF < _IBC)
                    def _():
                        pltpu.async_copy(
                            hs_hbm.at[ev.at[jb + _NBUF, 0]], bufs[b], sems[b]
                        )

    @pl.when(wid < _NW - 1)
    def _():
        agg_slab(eidx_hbm, wid * _BPW)

    @pl.when(wid == _NW - 1)
    def _():
        agg_slab(tail_hbm, 0)

    plsc.subcore_barrier()
    pltpu.sync_copy(
        acc_sh.at[pl.ds(s * _RPS, _RPS)],
        out_hbm.at[c].at[pl.ds(s * _RPS, _RPS)],
    )


_agg = functools.partial(
    pl.kernel,
    out_type=jax.ShapeDtypeStruct((_NC, _NPAD, _D), jnp.float32),
    mesh=_mesh,
    scratch_types=[
        pltpu.VMEM((_IBC, 2, _EB), jnp.int32),
        pltpu.VMEM((_EB, _D), jnp.float32),
        pltpu.VMEM((_EB, _D), jnp.float32),
        pltpu.SemaphoreType.DMA,
        pltpu.SemaphoreType.DMA,
        pltpu.VMEM_SHARED((_NPAD, _D), jnp.float32),
    ],
)(_agg_body)


def _dot(a, b):
    return lax.dot_general(
        a,
        b,
        (((1,), (0,)), ((), ())),
        precision=lax.Precision.HIGHEST,
        preferred_element_type=jnp.float32,
    )


def _dinv_col(degp_ref):
    deg = degp_ref[0, :, 0:1] + degp_ref[1, :, 0:1] + 1.0
    return lax.rsqrt(deg)


def _mm_body(x_ref, w_ref, o_ref):
    o_ref[...] = _dot(x_ref[...], w_ref[...])


def _scale_body(h_ref, degp_ref, hs_ref):
    hs_ref[...] = h_ref[...] * _dinv_col(degp_ref)


def _combine_mm_body(accp_ref, hs_ref, degp_ref, b_ref, w_ref, o_ref):
    d = _dinv_col(degp_ref)
    z = d * (accp_ref[0] + accp_ref[1] + hs_ref[...]) + b_ref[...]
    z = jnp.maximum(z, 0.0)
    o_ref[...] = _dot(z, w_ref[...]) * d


def _final_body(accp_ref, hs_ref, degp_ref, b_ref, o_ref):
    d = _dinv_col(degp_ref)
    o_ref[...] = d * (accp_ref[0] + accp_ref[1] + hs_ref[...]) + b_ref[...]


def kernel(x, edge_idx, W1, b1, W2, b2):
    n, d = x.shape
    grid = (n // _MBLK,)
    ei3 = jnp.transpose(edge_idx.reshape(2, _NRBLK, _EB), (1, 0, 2))
    tail = jnp.concatenate([ei3[(_NW - 1) * _BPW :], jnp.asarray(_PAD3)])

    degp = _deg(ei3, tail)

    h1 = pl.pallas_call(
        _mm_body,
        grid=grid,
        in_specs=[
            pl.BlockSpec((_MBLK, d), lambda i: (i, 0)),
            pl.BlockSpec((d, d), lambda i: (0, 0)),
        ],
        out_specs=pl.BlockSpec((_MBLK, d), lambda i: (i, 0)),
        out_shape=jax.ShapeDtypeStruct((n, d), jnp.float32),
    )(x, W1)

    hs1 = pl.pallas_call(
        _scale_body,
        grid=grid,
        in_specs=[
            pl.BlockSpec((_MBLK, d), lambda i: (i, 0)),
            pl.BlockSpec((_NC, _MBLK, 16), lambda i: (0, i, 0)),
        ],
        out_specs=pl.BlockSpec((_MBLK, d), lambda i: (i, 0)),
        out_shape=jax.ShapeDtypeStruct((n, d), jnp.float32),
    )(h1, degp)

    acc1 = _agg(hs1, ei3, tail)

    hs2 = pl.pallas_call(
        _combine_mm_body,
        grid=grid,
        in_specs=[
            pl.BlockSpec((_NC, _MBLK, _D), lambda i: (0, i, 0)),
            pl.BlockSpec((_MBLK, d), lambda i: (i, 0)),
            pl.BlockSpec((_NC, _MBLK, 16), lambda i: (0, i, 0)),
            pl.BlockSpec((1, d), lambda i: (0, 0)),
            pl.BlockSpec((d, d), lambda i: (0, 0)),
        ],
        out_specs=pl.BlockSpec((_MBLK, d), lambda i: (i, 0)),
        out_shape=jax.ShapeDtypeStruct((n, d), jnp.float32),
    )(acc1, hs1, degp, b1.reshape(1, d), W2)

    acc2 = _agg(hs2, ei3, tail)

    out = pl.pallas_call(
        _final_body,
        grid=grid,
        in_specs=[
            pl.BlockSpec((_NC, _MBLK, _D), lambda i: (0, i, 0)),
            pl.BlockSpec((_MBLK, d), lambda i: (i, 0)),
            pl.BlockSpec((_NC, _MBLK, 16), lambda i: (0, i, 0)),
            pl.BlockSpec((1, d), lambda i: (0, 0)),
        ],
        out_specs=pl.BlockSpec((_MBLK, d), lambda i: (i, 0)),
        out_shape=jax.ShapeDtypeStruct((n, d), jnp.float32),
    )(acc2, hs2, degp, b2.reshape(1, d))

    return out


# speedup vs baseline: 1.0797x; 1.0446x over previous
import dataclasses
import functools

import numpy as np

import jax
import jax.numpy as jnp
from jax import lax
from jax.experimental import pallas as pl
from jax.experimental.pallas import tpu as pltpu
from jax.experimental.pallas import tpu_sc as plsc

_N = 10000
_E = 320000
_D = 128
_NC = 2
_NS = 16
_NW = _NC * _NS
_EB = 128
_BPW = 80
_NBLK = _NW * _BPW
_EPAD = _NBLK * _EB
_NPAD = 10240
_RPS = _NPAD // _NS
_MBLK = 2000

_NRBLK = _E // _EB
_PAD3 = np.stack(
    [
        (np.arange(_EPAD - _E, dtype=np.int32) % 8192).reshape(_NBLK - _NRBLK, _EB),
        (_N + np.arange(_EPAD - _E, dtype=np.int32) % 240).reshape(_NBLK - _NRBLK, _EB),
    ],
    axis=1,
)

_mesh = plsc.VectorSubcoreMesh(
    core_axis_name="c", subcore_axis_name="s", num_cores=_NC, num_subcores=_NS
)


_DIBC = 16
_DNCH = _BPW // _DIBC


def _deg_body(dst_hbm, tail_hbm, out_hbm, dstv, hist_v, red_v, out16_v, sh):
    c = lax.axis_index("c")
    s = lax.axis_index("s")
    wid = s * _NC + c

    @pl.loop(0, _NPAD, step=16)
    def _(k):
        hist_v[pl.ds(k, 16)] = jnp.zeros((16,), jnp.int32)

    def hist_slab(src_ref, base):
        @pl.loop(0, _DNCH)
        def _(ci):
            pltpu.sync_copy(src_ref.at[pl.ds(base + ci * _DIBC, _DIBC)], dstv)

            @pl.loop(0, _DIBC)
            def _(j):
                for k in range(_EB // 16):
                    idx = dstv[j, 1, pl.ds(k * 16, 16)]
                    cnt, last = plsc.scan_count(idx)
                    plsc.addupdate_scatter(hist_v, [idx], cnt, mask=last)

    @pl.when(wid < _NW - 1)
    def _():
        hist_slab(dst_hbm, wid * _BPW)

    @pl.when(wid == _NW - 1)
    def _():
        hist_slab(tail_hbm, 0)

    pltpu.sync_copy(hist_v, sh.at[s])
    plsc.subcore_barrier()
    for k in range(_NS):
        pltpu.sync_copy(sh.at[k].at[pl.ds(s * _RPS, _RPS)], red_v.at[k])

    @pl.loop(0, _RPS, step=16)
    def _(l):
        acc = jnp.zeros((16,), jnp.int32)
        for k in range(_NS):
            acc = acc + red_v[k, pl.ds(l, 16)]
        cf = acc.astype(jnp.float32)
        for i in range(16):
            out16_v[l + i, :] = jnp.full((16,), cf[i], jnp.float32)

    pltpu.sync_copy(out16_v, out_hbm.at[c].at[pl.ds(s * _RPS, _RPS)])


_cp = pltpu.CompilerParams()
if "needs_layout_passes" in pltpu.CompilerParams.__dataclass_fields__:
    _cp = dataclasses.replace(_cp, needs_layout_passes=False)

_deg = functools.partial(
    pl.kernel,
    out_type=jax.ShapeDtypeStruct((_NC, _NPAD, 16), jnp.float32),
    compiler_params=_cp,
    mesh=_mesh,
    scratch_types=[
        pltpu.VMEM((_DIBC, 2, _EB), jnp.int32),
        pltpu.VMEM((_NPAD,), jnp.int32),
        pltpu.VMEM((_NS, _RPS), jnp.int32),
        pltpu.VMEM((_RPS, 16), jnp.float32),
        pltpu.VMEM_SHARED((_NS, _NPAD), jnp.int32),
    ],
)(_deg_body)


_NBUF = 2
_IBC = 40
_NCH = _BPW // _IBC


def _agg_body(
    hs_hbm, eidx_hbm, tail_hbm, out_hbm,
    ev, rows0, rows1, sem0, sem1, acc_sh
):
    c = lax.axis_index("c")
    s = lax.axis_index("s")
    wid = s * _NC + c

    @pl.loop(0, _EB)
    def _(i):
        for k in range(_D // 16):
            rows0[i, pl.ds(k * 16, 16)] = jnp.zeros((16,), jnp.float32)

    @pl.loop(0, _RPS // _EB)
    def _(i):
        pltpu.sync_copy(rows0, acc_sh.at[pl.ds(s * _RPS + i * _EB, _EB)])

    plsc.subcore_barrier()

    bufs = (rows0, rows1)
    sems = (sem0, sem1)

    def agg_slab(src_ref, base):
        @pl.loop(0, _NCH)
        def _(ci):
            cb = base + ci * _IBC
            pltpu.sync_copy(src_ref.at[pl.ds(cb, _IBC)], ev)
            for b in range(_NBUF):
                pltpu.async_copy(hs_hbm.at[ev.at[b, 0]], bufs[b], sems[b])

            @pl.loop(0, _IBC, step=_NBUF)
            def _(j):
                for b in range(_NBUF):
                    jb = j + b
                    pltpu.make_async_copy(
                        hs_hbm.at[ev.at[jb, 0]], bufs[b], sems[b]
                    ).wait()
                    pltpu.sync_copy(bufs[b], acc_sh.at[ev.at[jb, 1]], add=True)

                    @pl.when(jb + _NBUF < _IBC)
                    def _():
                        pltpu.async_copy(
                            hs_hbm.at[ev.at[jb + _NBUF, 0]], bufs[b], sems[b]
                        )

    @pl.when(wid < _NW - 1)
    def _():
        agg_slab(eidx_hbm, wid * _BPW)

    @pl.when(wid == _NW - 1)
    def _():
        agg_slab(tail_hbm, 0)

    plsc.subcore_barrier()
    pltpu.sync_copy(
        acc_sh.at[pl.ds(s * _RPS, _RPS)],
        out_hbm.at[c].at[pl.ds(s * _RPS, _RPS)],
    )


_agg = functools.partial(
    pl.kernel,
    out_type=jax.ShapeDtypeStruct((_NC, _NPAD, _D), jnp.float32),
    mesh=_mesh,
    scratch_types=[
        pltpu.VMEM((_IBC, 2, _EB), jnp.int32),
        pltpu.VMEM((_EB, _D), jnp.float32),
        pltpu.VMEM((_EB, _D), jnp.float32),
        pltpu.SemaphoreType.DMA,
        pltpu.SemaphoreType.DMA,
        pltpu.VMEM_SHARED((_NPAD, _D), jnp.float32),
    ],
)(_agg_body)


def _dot(a, b):
    return lax.dot_general(
        a,
        b,
        (((1,), (0,)), ((), ())),
        precision=lax.Precision.HIGHEST,
        preferred_element_type=jnp.float32,
    )


def _dinv_col(degp_ref):
    deg = degp_ref[0, :, 0:1] + degp_ref[1, :, 0:1] + 1.0
    return lax.rsqrt(deg)


def _mm_body(x_ref, w_ref, o_ref):
    o_ref[...] = _dot(x_ref[...], w_ref[...])


def _scale_body(h_ref, degp_ref, hs_ref):
    hs_ref[...] = h_ref[...] * _dinv_col(degp_ref)


def _combine_mm_body(accp_ref, hs_ref, degp_ref, b_ref, w_ref, o_ref):
    d = _dinv_col(degp_ref)
    z = d * (accp_ref[0] + accp_ref[1] + hs_ref[...]) + b_ref[...]
    z = jnp.maximum(z, 0.0)
    o_ref[...] = _dot(z, w_ref[...]) * d


def _final_body(accp_ref, hs_ref, degp_ref, b_ref, o_ref):
    d = _dinv_col(degp_ref)
    o_ref[...] = d * (accp_ref[0] + accp_ref[1] + hs_ref[...]) + b_ref[...]


def kernel(x, edge_idx, W1, b1, W2, b2):
    n, d = x.shape
    grid = (n // _MBLK,)
    ei3 = jnp.transpose(edge_idx.reshape(2, _NRBLK, _EB), (1, 0, 2))
    tail = jnp.concatenate([ei3[(_NW - 1) * _BPW :], jnp.asarray(_PAD3)])

    degp = _deg(ei3, tail)

    h1 = pl.pallas_call(
        _mm_body,
        grid=grid,
        in_specs=[
            pl.BlockSpec((_MBLK, d), lambda i: (i, 0)),
            pl.BlockSpec((d, d), lambda i: (0, 0)),
        ],
        out_specs=pl.BlockSpec((_MBLK, d), lambda i: (i, 0)),
        out_shape=jax.ShapeDtypeStruct((n, d), jnp.float32),
    )(x, W1)

    hs1 = pl.pallas_call(
        _scale_body,
        grid=grid,
        in_specs=[
            pl.BlockSpec((_MBLK, d), lambda i: (i, 0)),
            pl.BlockSpec((_NC, _MBLK, 16), lambda i: (0, i, 0)),
        ],
        out_specs=pl.BlockSpec((_MBLK, d), lambda i: (i, 0)),
        out_shape=jax.ShapeDtypeStruct((n, d), jnp.float32),
    )(h1, degp)

    acc1 = _agg(hs1, ei3, tail)

    hs2 = pl.pallas_call(
        _combine_mm_body,
        grid=grid,
        in_specs=[
            pl.BlockSpec((_NC, _MBLK, _D), lambda i: (0, i, 0)),
            pl.BlockSpec((_MBLK, d), lambda i: (i, 0)),
            pl.BlockSpec((_NC, _MBLK, 16), lambda i: (0, i, 0)),
            pl.BlockSpec((1, d), lambda i: (0, 0)),
            pl.BlockSpec((d, d), lambda i: (0, 0)),
        ],
        out_specs=pl.BlockSpec((_MBLK, d), lambda i: (i, 0)),
        out_shape=jax.ShapeDtypeStruct((n, d), jnp.float32),
    )(acc1, hs1, degp, b1.reshape(1, d), W2)

    acc2 = _agg(hs2, ei3, tail)

    out = pl.pallas_call(
        _final_body,
        grid=grid,
        in_specs=[
            pl.BlockSpec((_NC, _MBLK, _D), lambda i: (0, i, 0)),
            pl.BlockSpec((_MBLK, d), lambda i: (i, 0)),
            pl.BlockSpec((_NC, _MBLK, 16), lambda i: (0, i, 0)),
            pl.BlockSpec((1, d), lambda i: (0, 0)),
        ],
        out_specs=pl.BlockSpec((_MBLK, d), lambda i: (i, 0)),
        out_shape=jax.ShapeDtypeStruct((n, d), jnp.float32),
    )(acc2, hs2, degp, b2.reshape(1, d))

    return out


# speedup vs baseline: 1.0872x; 1.0070x over previous
import dataclasses
import functools

import numpy as np

import jax
import jax.numpy as jnp
from jax import lax
from jax.experimental import pallas as pl
from jax.experimental.pallas import tpu as pltpu
from jax.experimental.pallas import tpu_sc as plsc

_N = 10000
_E = 320000
_D = 128
_NC = 2
_NS = 16
_NW = _NC * _NS
_EB = 128
_BPW = 80
_NBLK = _NW * _BPW
_EPAD = _NBLK * _EB
_NPAD = 10240
_RPS = _NPAD // _NS
_MBLK = 2000

_NRBLK = _E // _EB
_PAD3 = np.stack(
    [
        (np.arange(_EPAD - _E, dtype=np.int32) % 8192).reshape(_NBLK - _NRBLK, _EB),
        (_N + np.arange(_EPAD - _E, dtype=np.int32) % 240).reshape(_NBLK - _NRBLK, _EB),
    ],
    axis=1,
)

_mesh = plsc.VectorSubcoreMesh(
    core_axis_name="c", subcore_axis_name="s", num_cores=_NC, num_subcores=_NS
)


_DIBC = 40
_DNCH = _BPW // _DIBC


def _deg_body(dst_hbm, tail_hbm, out_hbm, dstv, hist_v, red_v, out16_v, sh):
    c = lax.axis_index("c")
    s = lax.axis_index("s")
    wid = s * _NC + c

    @pl.loop(0, _NPAD, step=16)
    def _(k):
        hist_v[pl.ds(k, 16)] = jnp.zeros((16,), jnp.int32)

    def hist_slab(src_ref, base):
        @pl.loop(0, _DNCH)
        def _(ci):
            pltpu.sync_copy(src_ref.at[pl.ds(base + ci * _DIBC, _DIBC)], dstv)

            @pl.loop(0, _DIBC)
            def _(j):
                for k in range(_EB // 16):
                    idx = dstv[j, 1, pl.ds(k * 16, 16)]
                    cnt, last = plsc.scan_count(idx)
                    plsc.addupdate_scatter(hist_v, [idx], cnt, mask=last)

    @pl.when(wid < _NW - 1)
    def _():
        hist_slab(dst_hbm, wid * _BPW)

    @pl.when(wid == _NW - 1)
    def _():
        hist_slab(tail_hbm, 0)

    pltpu.sync_copy(hist_v, sh.at[s])
    plsc.subcore_barrier()
    for k in range(_NS):
        pltpu.sync_copy(sh.at[k].at[pl.ds(s * _RPS, _RPS)], red_v.at[k])

    @pl.loop(0, _RPS, step=16)
    def _(l):
        acc = jnp.zeros((16,), jnp.int32)
        for k in range(_NS):
            acc = acc + red_v[k, pl.ds(l, 16)]
        cf = acc.astype(jnp.float32)
        for i in range(16):
            out16_v[l + i, :] = jnp.full((16,), cf[i], jnp.float32)

    pltpu.sync_copy(out16_v, out_hbm.at[c].at[pl.ds(s * _RPS, _RPS)])


_cp = pltpu.CompilerParams()
if "needs_layout_passes" in pltpu.CompilerParams.__dataclass_fields__:
    _cp = dataclasses.replace(_cp, needs_layout_passes=False)

_deg = functools.partial(
    pl.kernel,
    out_type=jax.ShapeDtypeStruct((_NC, _NPAD, 16), jnp.float32),
    compiler_params=_cp,
    mesh=_mesh,
    scratch_types=[
        pltpu.VMEM((_DIBC, 2, _EB), jnp.int32),
        pltpu.VMEM((_NPAD,), jnp.int32),
        pltpu.VMEM((_NS, _RPS), jnp.int32),
        pltpu.VMEM((_RPS, 16), jnp.float32),
        pltpu.VMEM_SHARED((_NS, _NPAD), jnp.int32),
    ],
)(_deg_body)


_NBUF = 2
_IBC = 40
_NCH = _BPW // _IBC


def _agg_body(
    hs_hbm, eidx_hbm, tail_hbm, out_hbm,
    ev, rows0, rows1, sem0, sem1, acc_sh
):
    c = lax.axis_index("c")
    s = lax.axis_index("s")
    wid = s * _NC + c

    @pl.loop(0, _EB)
    def _(i):
        for k in range(_D // 16):
            rows0[i, pl.ds(k * 16, 16)] = jnp.zeros((16,), jnp.float32)

    @pl.loop(0, _RPS // _EB)
    def _(i):
        pltpu.sync_copy(rows0, acc_sh.at[pl.ds(s * _RPS + i * _EB, _EB)])

    plsc.subcore_barrier()

    bufs = (rows0, rows1)
    sems = (sem0, sem1)

    def agg_slab(src_ref, base):
        @pl.loop(0, _NCH)
        def _(ci):
            cb = base + ci * _IBC
            pltpu.sync_copy(src_ref.at[pl.ds(cb, _IBC)], ev)
            for b in range(_NBUF):
                pltpu.async_copy(hs_hbm.at[ev.at[b, 0]], bufs[b], sems[b])

            @pl.loop(0, _IBC, step=_NBUF)
            def _(j):
                for b in range(_NBUF):
                    jb = j + b
                    pltpu.make_async_copy(
                        hs_hbm.at[ev.at[jb, 0]], bufs[b], sems[b]
                    ).wait()
                    pltpu.sync_copy(bufs[b], acc_sh.at[ev.at[jb, 1]], add=True)

                    @pl.when(jb + _NBUF < _IBC)
                    def _():
                        pltpu.async_copy(
                            hs_hbm.at[ev.at[jb + _NBUF, 0]], bufs[b], sems[b]
                        )

    @pl.when(wid < _NW - 1)
    def _():
        agg_slab(eidx_hbm, wid * _BPW)

    @pl.when(wid == _NW - 1)
    def _():
        agg_slab(tail_hbm, 0)

    plsc.subcore_barrier()
    pltpu.sync_copy(
        acc_sh.at[pl.ds(s * _RPS, _RPS)],
        out_hbm.at[c].at[pl.ds(s * _RPS, _RPS)],
    )


_agg = functools.partial(
    pl.kernel,
    out_type=jax.ShapeDtypeStruct((_NC, _NPAD, _D), jnp.float32),
    mesh=_mesh,
    scratch_types=[
        pltpu.VMEM((_IBC, 2, _EB), jnp.int32),
        pltpu.VMEM((_EB, _D), jnp.float32),
        pltpu.VMEM((_EB, _D), jnp.float32),
        pltpu.SemaphoreType.DMA,
        pltpu.SemaphoreType.DMA,
        pltpu.VMEM_SHARED((_NPAD, _D), jnp.float32),
    ],
)(_agg_body)


def _dot(a, b):
    return lax.dot_general(
        a,
        b,
        (((1,), (0,)), ((), ())),
        precision=lax.Precision.HIGHEST,
        preferred_element_type=jnp.float32,
    )


def _dinv_col(degp_ref):
    deg = degp_ref[0, :, 0:1] + degp_ref[1, :, 0:1] + 1.0
    return lax.rsqrt(deg)


def _mm_body(x_ref, w_ref, o_ref):
    o_ref[...] = _dot(x_ref[...], w_ref[...])


def _scale_body(h_ref, degp_ref, hs_ref):
    hs_ref[...] = h_ref[...] * _dinv_col(degp_ref)


def _combine_mm_body(accp_ref, hs_ref, degp_ref, b_ref, w_ref, o_ref):
    d = _dinv_col(degp_ref)
    z = d * (accp_ref[0] + accp_ref[1] + hs_ref[...]) + b_ref[...]
    z = jnp.maximum(z, 0.0)
    o_ref[...] = _dot(z, w_ref[...]) * d


def _final_body(accp_ref, hs_ref, degp_ref, b_ref, o_ref):
    d = _dinv_col(degp_ref)
    o_ref[...] = d * (accp_ref[0] + accp_ref[1] + hs_ref[...]) + b_ref[...]


def kernel(x, edge_idx, W1, b1, W2, b2):
    n, d = x.shape
    grid = (n // _MBLK,)
    ei3 = jnp.transpose(edge_idx.reshape(2, _NRBLK, _EB), (1, 0, 2))
    tail = jnp.concatenate([ei3[(_NW - 1) * _BPW :], jnp.asarray(_PAD3)])

    degp = _deg(ei3, tail)

    h1 = pl.pallas_call(
        _mm_body,
        grid=grid,
        in_specs=[
            pl.BlockSpec((_MBLK, d), lambda i: (i, 0)),
            pl.BlockSpec((d, d), lambda i: (0, 0)),
        ],
        out_specs=pl.BlockSpec((_MBLK, d), lambda i: (i, 0)),
        out_shape=jax.ShapeDtypeStruct((n, d), jnp.float32),
    )(x, W1)

    hs1 = pl.pallas_call(
        _scale_body,
        grid=grid,
        in_specs=[
            pl.BlockSpec((_MBLK, d), lambda i: (i, 0)),
            pl.BlockSpec((_NC, _MBLK, 16), lambda i: (0, i, 0)),
        ],
        out_specs=pl.BlockSpec((_MBLK, d), lambda i: (i, 0)),
        out_shape=jax.ShapeDtypeStruct((n, d), jnp.float32),
    )(h1, degp)

    acc1 = _agg(hs1, ei3, tail)

    hs2 = pl.pallas_call(
        _combine_mm_body,
        grid=grid,
        in_specs=[
            pl.BlockSpec((_NC, _MBLK, _D), lambda i: (0, i, 0)),
            pl.BlockSpec((_MBLK, d), lambda i: (i, 0)),
            pl.BlockSpec((_NC, _MBLK, 16), lambda i: (0, i, 0)),
            pl.BlockSpec((1, d), lambda i: (0, 0)),
            pl.BlockSpec((d, d), lambda i: (0, 0)),
        ],
        out_specs=pl.BlockSpec((_MBLK, d), lambda i: (i, 0)),
        out_shape=jax.ShapeDtypeStruct((n, d), jnp.float32),
    )(acc1, hs1, degp, b1.reshape(1, d), W2)

    acc2 = _agg(hs2, ei3, tail)

    out = pl.pallas_call(
        _final_body,
        grid=grid,
        in_specs=[
            pl.BlockSpec((_NC, _MBLK, _D), lambda i: (0, i, 0)),
            pl.BlockSpec((_MBLK, d), lambda i: (i, 0)),
            pl.BlockSpec((_NC, _MBLK, 16), lambda i: (0, i, 0)),
            pl.BlockSpec((1, d), lambda i: (0, 0)),
        ],
        out_specs=pl.BlockSpec((_MBLK, d), lambda i: (i, 0)),
        out_shape=jax.ShapeDtypeStruct((n, d), jnp.float32),
    )(acc2, hs2, degp, b2.reshape(1, d))

    return out
